# baseline (speedup 1.0000x reference)
import functools

import numpy as np
import jax
import jax.numpy as jnp
from jax import lax
from jax.experimental import pallas as pl
from jax.experimental.pallas import tpu as pltpu
from jax.experimental.pallas import tpu_sc as plsc

_N_SEG = 64


def kernel(inp):
    b, n_t, d = inp.shape
    rows = b * _N_SEG

    info = plsc.get_sparse_core_info()
    num_workers = info.num_cores * info.num_subcores
    rpw = rows // num_workers
    wpb = _N_SEG // rpw

    table = inp.reshape(b * n_t, d)
    mesh = plsc.VectorSubcoreMesh(core_axis_name="c", subcore_axis_name="s")

    @functools.partial(
        pl.kernel,
        mesh=mesh,
        out_type=jax.ShapeDtypeStruct((b, _N_SEG, d), jnp.float32),
        scratch_types=[
            pltpu.VMEM((rpw,), jnp.int32),
            pltpu.VMEM((rpw, d), jnp.float32),
            pltpu.SemaphoreType.DMA,
        ],
    )
    def gather_rows(table_hbm, idx_hbm, out_hbm, idx_v, rows_v, sem):
        wid = lax.axis_index("s") * info.num_cores + lax.axis_index("c")
        base = wid * rpw
        pltpu.sync_copy(idx_hbm.at[pl.ds(base, rpw)], idx_v)
        pltpu.async_copy(table_hbm.at[idx_v], rows_v, sem).wait()
        out_b = wid // wpb
        out_k = (wid % wpb) * rpw
        pltpu.sync_copy(rows_v, out_hbm.at[out_b, pl.ds(out_k, rpw)])

    t_vec = np.linspace(1, n_t, _N_SEG + 1)
    starts = [int(round(x)) - 1 for x in t_vec[:-1]]
    flat_idx = np.asarray(
        [bi * n_t + t for bi in range(b) for t in starts], dtype=np.int32
    )
    return gather_rows(table, jnp.asarray(flat_idx))
